# baseline (speedup 1.0000x reference)
import functools

import jax
import jax.numpy as jnp
from jax import lax
from jax.experimental import pallas as pl
from jax.experimental.pallas import tpu as pltpu
from jax.experimental.pallas import tpu_sc as plsc

N = 10000
E = 160000
IN_DIM = 256
HID = 256
NCLS = 128
H0 = 4
F0 = H0 * HID
F1 = NCLS

NC = 2
NS = 16
NW = NC * NS
EPW = E // NW
GCH = 40
IB = EPW + 64
SCH = 32

_f32 = jnp.float32
_i32 = jnp.int32


def _mesh():
    return plsc.VectorSubcoreMesh(core_axis_name="c", subcore_axis_name="s")


def _mm2_body(x_ref, ws_ref, wd_ref, fs_ref, fd_ref):
    xv = x_ref[...]
    fs_ref[...] = jnp.dot(xv, ws_ref[...], preferred_element_type=_f32)
    fd_ref[...] = jnp.dot(xv, wd_ref[...], preferred_element_type=_f32)


def _mm2(x, ws, wd):
    k, f = x.shape[1], ws.shape[1]
    nb = 10
    rb = N // nb
    return pl.pallas_call(
        _mm2_body,
        grid=(nb,),
        in_specs=[
            pl.BlockSpec((rb, k), lambda i: (i, 0)),
            pl.BlockSpec((k, f), lambda i: (0, 0)),
            pl.BlockSpec((k, f), lambda i: (0, 0)),
        ],
        out_specs=[
            pl.BlockSpec((rb, f), lambda i: (i, 0)),
            pl.BlockSpec((rb, f), lambda i: (i, 0)),
        ],
        out_shape=[
            jax.ShapeDtypeStruct((N, f), _f32),
            jax.ShapeDtypeStruct((N, f), _f32),
        ],
    )(x, ws, wd)


def _sc_gather(fs, fd, src, dst):
    f = fs.shape[1]

    @functools.partial(
        pl.kernel,
        out_type=(
            jax.ShapeDtypeStruct((E, f), _f32),
            jax.ShapeDtypeStruct((E, f), _f32),
        ),
        mesh=_mesh(),
        scratch_types=[
            pltpu.VMEM((GCH,), _i32),
            pltpu.VMEM((GCH,), _i32),
            pltpu.VMEM((GCH, f), _f32),
            pltpu.VMEM((GCH, f), _f32),
            pltpu.SemaphoreType.DMA,
            pltpu.SemaphoreType.DMA,
        ],
    )
    def k(fs_hbm, fd_hbm, src_hbm, dst_hbm, s_hbm, d_hbm,
          idx_s, idx_d, rows_s, rows_d, sem_s, sem_d):
        wid = lax.axis_index("s") * NC + lax.axis_index("c")
        base = wid * EPW

        @pl.loop(0, EPW, step=GCH)
        def _(off):
            o = base + off
            pltpu.sync_copy(src_hbm.at[pl.ds(o, GCH)], idx_s)
            pltpu.sync_copy(dst_hbm.at[pl.ds(o, GCH)], idx_d)
            cs = pltpu.async_copy(fs_hbm.at[idx_s], rows_s, sem_s)
            cd = pltpu.async_copy(fd_hbm.at[idx_d], rows_d, sem_d)
            cs.wait()
            cd.wait()
            pltpu.sync_copy(rows_s, s_hbm.at[pl.ds(o, GCH)])
            pltpu.sync_copy(rows_d, d_hbm.at[pl.ds(o, GCH)])

    return k(fs, fd, src, dst)


def _edge_body(nh, fh, s_ref, d_ref, a_ref, c_ref):
    s = s_ref[...]
    d = d_ref[...]
    a = a_ref[...]
    f = nh * fh
    c_ref[:, f:f + 128] = jnp.zeros_like(c_ref[:, f:f + 128])
    for h in range(nh):
        sh = s[:, h * fh:(h + 1) * fh]
        dh = d[:, h * fh:(h + 1) * fh]
        ah = a[:, h * fh:(h + 1) * fh]
        z = sh + dh
        lr = jnp.where(z > 0, z, 0.2 * z)
        lg = jnp.sum(lr * ah, axis=1, keepdims=True)
        ex = jnp.exp(jnp.minimum(lg, 80.0))
        c_ref[:, h * fh:(h + 1) * fh] = sh * ex
        c_ref[:, f + h:f + h + 1] = ex


def _edge_math(s, d, a_flat, nh):
    f = s.shape[1]
    fh = f // nh
    eb = 640
    nb = E // eb
    return pl.pallas_call(
        functools.partial(_edge_body, nh, fh),
        grid=(nb,),
        in_specs=[
            pl.BlockSpec((eb, f), lambda i: (i, 0)),
            pl.BlockSpec((eb, f), lambda i: (i, 0)),
            pl.BlockSpec((1, f), lambda i: (0, 0)),
        ],
        out_specs=pl.BlockSpec((eb, f + 128), lambda i: (i, 0)),
        out_shape=jax.ShapeDtypeStruct((E, f + 128), _f32),
    )(s, d, a_flat)


NP_ = 10240
ECH = 4000


def _sc_scatter(c, dst, z, rng, sweeps):
    f = c.shape[1]
    accw = (rng + 1) * f
    cp = pltpu.CompilerParams()
    if "needs_layout_passes" in pltpu.CompilerParams.__dataclass_fields__:
        import dataclasses as _dc
        cp = _dc.replace(cp, needs_layout_passes=False)

    @functools.partial(
        pl.kernel,
        out_type=jax.ShapeDtypeStruct((NP_ * f,), _f32),
        mesh=_mesh(),
        compiler_params=cp,
        scratch_types=[
            pltpu.VMEM((ECH,), _i32),
            pltpu.VMEM((ECH + 64,), _i32),
            pltpu.VMEM((ECH + 64,), _i32),
            pltpu.VMEM((16, f), _f32),
            pltpu.VMEM((16, f), _f32),
            pltpu.VMEM((accw,), _f32),
            pltpu.SemaphoreType.DMA,
            pltpu.SemaphoreType.DMA,
        ],
    )
    def k(c_hbm, dst_hbm, z_hbm, out_hbm, dch, ids, lidx, rb0, rb1, acc,
          sem0, sem1):
        cid = lax.axis_index("c")
        sid = lax.axis_index("s")
        wid = sid * NC + cid
        iota = lax.iota(_i32, 16)
        zvec = jnp.zeros((16,), _i32)
        tvec = jnp.full((16,), rng, _i32)

        for t in range(sweeps):
            base = (t * NW + wid) * rng
            pltpu.sync_copy(z_hbm, acc)

            @pl.loop(0, E, step=ECH)
            def _(co):
                pltpu.sync_copy(dst_hbm.at[pl.ds(co, ECH)], dch)

                def scan_body(j, cnt):
                    dv = dch[pl.ds(j * 16, 16)]
                    inr = (dv >= base) & (dv < base + rng)
                    incl = plsc.cumsum(inr.astype(_i32))
                    pos = cnt + incl - 1
                    plsc.store_scatter(ids, [pos], co + j * 16 + iota,
                                       mask=inr)
                    plsc.store_scatter(lidx, [pos], dv - base, mask=inr)
                    return cnt + plsc.all_reduce_population_count(inr)

                cnt = lax.fori_loop(0, ECH // 16, scan_body,
                                    jnp.zeros((16,), _i32))
                kk = cnt[0]
                for pv in range(3):
                    plsc.store_scatter(ids, [kk + pv * 16 + iota], zvec)
                    plsc.store_scatter(lidx, [kk + pv * 16 + iota], tvec)

                def acc_chunk(off16, rbuf):
                    lvec = lidx[pl.ds(off16, 16)]
                    ros = [lvec[r] * f for r in range(16)]

                    @pl.loop(0, f, step=16)
                    def _(cc):
                        for r in range(16):
                            plsc.addupdate(acc.at[pl.ds(ros[r] + cc, 16)],
                                           rbuf[r, pl.ds(cc, 16)])

                mm = (kk + 31) // 32
                pltpu.async_copy(c_hbm.at[ids.at[pl.ds(0, 16)]], rb0, sem0)

                def pair_body(q, _):
                    o1 = q * 32 + 16
                    pltpu.async_copy(c_hbm.at[ids.at[pl.ds(o1, 16)]],
                                     rb1, sem1)
                    pltpu.make_async_copy(
                        c_hbm.at[ids.at[pl.ds(0, 16)]], rb0, sem0).wait()
                    acc_chunk(q * 32, rb0)
                    pltpu.async_copy(
                        c_hbm.at[ids.at[pl.ds(o1 + 16, 16)]], rb0, sem0)
                    pltpu.make_async_copy(
                        c_hbm.at[ids.at[pl.ds(0, 16)]], rb1, sem1).wait()
                    acc_chunk(o1, rb1)
                    return 0

                lax.fori_loop(0, mm, pair_body, 0)
                pltpu.make_async_copy(
                    c_hbm.at[ids.at[pl.ds(0, 16)]], rb0, sem0).wait()

            pltpu.sync_copy(acc.at[pl.ds(0, rng * f)],
                            out_hbm.at[pl.ds(base * f, rng * f)])

    return k(c, dst, z).reshape(NP_, f)


def _fin0_body(o_ref, b_ref, ws_ref, wd_ref, fs_ref, fd_ref):
    ov = o_ref[...]
    b = b_ref[...]
    cols = []
    for h in range(H0):
        oh = ov[:, h * HID:(h + 1) * HID]
        bh = b[:, h * HID:(h + 1) * HID]
        dh = ov[:, F0 + h:F0 + h + 1]
        hh = oh / (dh + 1e-16) + bh
        cols.append(jnp.where(hh > 0, hh, jnp.exp(jnp.minimum(hh, 0.0)) - 1.0))
    hv = jnp.concatenate(cols, axis=1)
    fs_ref[...] = jnp.dot(hv, ws_ref[...], preferred_element_type=_f32)
    fd_ref[...] = jnp.dot(hv, wd_ref[...], preferred_element_type=_f32)


def _fin0(out0, b0f, w1s, w1d):
    nb = 10
    rb = N // nb
    return pl.pallas_call(
        _fin0_body,
        grid=(nb,),
        in_specs=[
            pl.BlockSpec((rb, F0 + 128), lambda i: (i, 0)),
            pl.BlockSpec((1, F0), lambda i: (0, 0)),
            pl.BlockSpec((F0, F1), lambda i: (0, 0)),
            pl.BlockSpec((F0, F1), lambda i: (0, 0)),
        ],
        out_specs=[
            pl.BlockSpec((rb, F1), lambda i: (i, 0)),
            pl.BlockSpec((rb, F1), lambda i: (i, 0)),
        ],
        out_shape=[
            jax.ShapeDtypeStruct((N, F1), _f32),
            jax.ShapeDtypeStruct((N, F1), _f32),
        ],
    )(out0, b0f, w1s, w1d)


def _fin1_body(o_ref, b_ref, out_ref):
    ov = o_ref[...]
    out_ref[...] = (ov[:, :F1] / (ov[:, F1:F1 + 1] + 1e-16)) + b_ref[...]


def _fin1(out1, b1f):
    nb = 10
    rb = N // nb
    return pl.pallas_call(
        _fin1_body,
        grid=(nb,),
        in_specs=[
            pl.BlockSpec((rb, F1 + 128), lambda i: (i, 0)),
            pl.BlockSpec((1, F1), lambda i: (0, 0)),
        ],
        out_specs=pl.BlockSpec((rb, F1), lambda i: (i, 0)),
        out_shape=jax.ShapeDtypeStruct((N, F1), _f32),
    )(out1, b1f)


def kernel(x, edge_index, W0_src, W0_dst, a0, b0, W1_src, W1_dst, a1, b1):
    src = edge_index[0].astype(_i32)
    dst = edge_index[1].astype(_i32)
    a0f = a0.reshape(1, F0)
    b0f = b0.reshape(1, F0)
    a1f = a1.reshape(1, F1)
    b1f = b1.reshape(1, F1)

    fs0, fd0 = _mm2(x, W0_src, W0_dst)
    s0, d0 = _sc_gather(fs0, fd0, src, dst)
    c0 = _edge_math(s0, d0, a0f, H0)
    z0 = jnp.zeros(((64 + 1) * (F0 + 128),), _f32)
    out0 = _sc_scatter(c0, dst, z0, rng=64, sweeps=5)

    fs1, fd1 = _fin0(out0, b0f, W1_src, W1_dst)
    s1, d1 = _sc_gather(fs1, fd1, src, dst)
    c1 = _edge_math(s1, d1, a1f, 1)
    z1 = jnp.zeros(((320 + 1) * (F1 + 128),), _f32)
    out1 = _sc_scatter(c1, dst, z1, rng=320, sweeps=1)

    return _fin1(out1, b1f)


# speedup vs baseline: 1.6456x; 1.6456x over previous
import functools

import jax
import jax.numpy as jnp
from jax import lax
from jax.experimental import pallas as pl
from jax.experimental.pallas import tpu as pltpu
from jax.experimental.pallas import tpu_sc as plsc

N = 10000
E = 160000
IN_DIM = 256
HID = 256
NCLS = 128
H0 = 4
F0 = H0 * HID
F1 = NCLS

NC = 2
NS = 16
NW = NC * NS
EPW = E // NW
GCH = 40
IB = EPW + 64
SCH = 32

_f32 = jnp.float32
_i32 = jnp.int32


def _mesh():
    return plsc.VectorSubcoreMesh(core_axis_name="c", subcore_axis_name="s")


def _mm2_body(x_ref, ws_ref, wd_ref, fs_ref, fd_ref):
    xv = x_ref[...]
    fs_ref[...] = jnp.dot(xv, ws_ref[...], preferred_element_type=_f32)
    fd_ref[...] = jnp.dot(xv, wd_ref[...], preferred_element_type=_f32)


def _mm2(x, ws, wd):
    k, f = x.shape[1], ws.shape[1]
    nb = 10
    rb = N // nb
    return pl.pallas_call(
        _mm2_body,
        grid=(nb,),
        in_specs=[
            pl.BlockSpec((rb, k), lambda i: (i, 0)),
            pl.BlockSpec((k, f), lambda i: (0, 0)),
            pl.BlockSpec((k, f), lambda i: (0, 0)),
        ],
        out_specs=[
            pl.BlockSpec((rb, f), lambda i: (i, 0)),
            pl.BlockSpec((rb, f), lambda i: (i, 0)),
        ],
        out_shape=[
            jax.ShapeDtypeStruct((N, f), _f32),
            jax.ShapeDtypeStruct((N, f), _f32),
        ],
    )(x, ws, wd)


def _sc_gather(fs, fd, src, dst):
    f = fs.shape[1]

    @functools.partial(
        pl.kernel,
        out_type=(
            jax.ShapeDtypeStruct((E, f), _f32),
            jax.ShapeDtypeStruct((E, f), _f32),
        ),
        mesh=_mesh(),
        scratch_types=[
            pltpu.VMEM((GCH,), _i32),
            pltpu.VMEM((GCH,), _i32),
            pltpu.VMEM((GCH, f), _f32),
            pltpu.VMEM((GCH, f), _f32),
            pltpu.SemaphoreType.DMA,
            pltpu.SemaphoreType.DMA,
        ],
    )
    def k(fs_hbm, fd_hbm, src_hbm, dst_hbm, s_hbm, d_hbm,
          idx_s, idx_d, rows_s, rows_d, sem_s, sem_d):
        wid = lax.axis_index("s") * NC + lax.axis_index("c")
        base = wid * EPW

        @pl.loop(0, EPW, step=GCH)
        def _(off):
            o = base + off
            pltpu.sync_copy(src_hbm.at[pl.ds(o, GCH)], idx_s)
            pltpu.sync_copy(dst_hbm.at[pl.ds(o, GCH)], idx_d)
            cs = pltpu.async_copy(fs_hbm.at[idx_s], rows_s, sem_s)
            cd = pltpu.async_copy(fd_hbm.at[idx_d], rows_d, sem_d)
            cs.wait()
            cd.wait()
            pltpu.sync_copy(rows_s, s_hbm.at[pl.ds(o, GCH)])
            pltpu.sync_copy(rows_d, d_hbm.at[pl.ds(o, GCH)])

    return k(fs, fd, src, dst)


def _edge_body(nh, fh, s_ref, d_ref, a_ref, c_ref):
    s = s_ref[...]
    d = d_ref[...]
    a = a_ref[...]
    f = nh * fh
    c_ref[:, f:f + 128] = jnp.zeros_like(c_ref[:, f:f + 128])
    for h in range(nh):
        sh = s[:, h * fh:(h + 1) * fh]
        dh = d[:, h * fh:(h + 1) * fh]
        ah = a[:, h * fh:(h + 1) * fh]
        z = sh + dh
        lr = jnp.where(z > 0, z, 0.2 * z)
        lg = jnp.sum(lr * ah, axis=1, keepdims=True)
        ex = jnp.exp(jnp.minimum(lg, 80.0))
        c_ref[:, h * fh:(h + 1) * fh] = sh * ex
        c_ref[:, f + h:f + h + 1] = ex


def _edge_math(s, d, a_flat, nh):
    f = s.shape[1]
    fh = f // nh
    eb = 640
    nb = E // eb
    return pl.pallas_call(
        functools.partial(_edge_body, nh, fh),
        grid=(nb,),
        in_specs=[
            pl.BlockSpec((eb, f), lambda i: (i, 0)),
            pl.BlockSpec((eb, f), lambda i: (i, 0)),
            pl.BlockSpec((1, f), lambda i: (0, 0)),
        ],
        out_specs=pl.BlockSpec((eb, f + 128), lambda i: (i, 0)),
        out_shape=jax.ShapeDtypeStruct((E, f + 128), _f32),
    )(s, d, a_flat)


NP_ = 10240
ECH = 4000


def _sc_scatter(c, dst, z, rng, sweeps):
    f = c.shape[1]
    accw = (rng + 1) * f
    cp = pltpu.CompilerParams()
    if "needs_layout_passes" in pltpu.CompilerParams.__dataclass_fields__:
        import dataclasses as _dc
        cp = _dc.replace(cp, needs_layout_passes=False)

    @functools.partial(
        pl.kernel,
        out_type=jax.ShapeDtypeStruct((NP_ * f,), _f32),
        mesh=_mesh(),
        compiler_params=cp,
        scratch_types=[
            pltpu.VMEM((ECH,), _i32),
            pltpu.VMEM((ECH + 64,), _i32),
            pltpu.VMEM((ECH + 64,), _i32),
            pltpu.VMEM((16, f), _f32),
            pltpu.VMEM((16, f), _f32),
            pltpu.VMEM((accw,), _f32),
            pltpu.SemaphoreType.DMA,
            pltpu.SemaphoreType.DMA,
        ],
    )
    def k(c_hbm, dst_hbm, z_hbm, out_hbm, dch, ids, lidx, rb0, rb1, acc,
          sem0, sem1):
        cid = lax.axis_index("c")
        sid = lax.axis_index("s")
        wid = sid * NC + cid
        iota = lax.iota(_i32, 16)
        zvec = jnp.zeros((16,), _i32)
        tvec = jnp.full((16,), rng, _i32)

        for t in range(sweeps):
            base = (t * NW + wid) * rng
            pltpu.sync_copy(z_hbm, acc)

            @pl.loop(0, E, step=ECH)
            def _(co):
                pltpu.sync_copy(dst_hbm.at[pl.ds(co, ECH)], dch)

                def scan_body(j, cnt):
                    dv = dch[pl.ds(j * 16, 16)]
                    inr = (dv >= base) & (dv < base + rng)
                    incl = plsc.cumsum(inr.astype(_i32))
                    pos = cnt + incl - 1
                    plsc.store_scatter(ids, [pos], co + j * 16 + iota,
                                       mask=inr)
                    plsc.store_scatter(lidx, [pos], dv - base, mask=inr)
                    return cnt + plsc.all_reduce_population_count(inr)

                cnt = lax.fori_loop(0, ECH // 16, scan_body,
                                    jnp.zeros((16,), _i32))
                kk = cnt[0]
                for pv in range(3):
                    plsc.store_scatter(ids, [kk + pv * 16 + iota], zvec)
                    plsc.store_scatter(lidx, [kk + pv * 16 + iota], tvec)

                def acc_chunk(off16, rbuf):
                    lvec = lidx[pl.ds(off16, 16)]
                    ros = [lvec[r] * f for r in range(16)]

                    @pl.loop(0, f, step=16)
                    def _(cc):
                        for r in range(16):
                            plsc.addupdate(acc.at[pl.ds(ros[r] + cc, 16)],
                                           rbuf[r, pl.ds(cc, 16)])

                mm = (kk + 31) // 32

                def pair_body(q, _):
                    o0 = q * 32
                    g0 = pltpu.async_copy(
                        c_hbm.at[ids.at[pl.ds(o0, 16)]], rb0, sem0)
                    g1 = pltpu.async_copy(
                        c_hbm.at[ids.at[pl.ds(o0 + 16, 16)]], rb1, sem1)
                    g0.wait()
                    acc_chunk(o0, rb0)
                    g1.wait()
                    acc_chunk(o0 + 16, rb1)
                    return 0

                lax.fori_loop(0, mm, pair_body, 0)

            pltpu.sync_copy(acc.at[pl.ds(0, rng * f)],
                            out_hbm.at[pl.ds(base * f, rng * f)])

    return k(c, dst, z).reshape(NP_, f)


def _fin0_body(o_ref, b_ref, ws_ref, wd_ref, fs_ref, fd_ref):
    ov = o_ref[...]
    b = b_ref[...]
    cols = []
    for h in range(H0):
        oh = ov[:, h * HID:(h + 1) * HID]
        bh = b[:, h * HID:(h + 1) * HID]
        dh = ov[:, F0 + h:F0 + h + 1]
        hh = oh / (dh + 1e-16) + bh
        cols.append(jnp.where(hh > 0, hh, jnp.exp(jnp.minimum(hh, 0.0)) - 1.0))
    hv = jnp.concatenate(cols, axis=1)
    fs_ref[...] = jnp.dot(hv, ws_ref[...], preferred_element_type=_f32)
    fd_ref[...] = jnp.dot(hv, wd_ref[...], preferred_element_type=_f32)


def _fin0(out0, b0f, w1s, w1d):
    nb = 10
    rb = N // nb
    return pl.pallas_call(
        _fin0_body,
        grid=(nb,),
        in_specs=[
            pl.BlockSpec((rb, F0 + 128), lambda i: (i, 0)),
            pl.BlockSpec((1, F0), lambda i: (0, 0)),
            pl.BlockSpec((F0, F1), lambda i: (0, 0)),
            pl.BlockSpec((F0, F1), lambda i: (0, 0)),
        ],
        out_specs=[
            pl.BlockSpec((rb, F1), lambda i: (i, 0)),
            pl.BlockSpec((rb, F1), lambda i: (i, 0)),
        ],
        out_shape=[
            jax.ShapeDtypeStruct((N, F1), _f32),
            jax.ShapeDtypeStruct((N, F1), _f32),
        ],
    )(out0, b0f, w1s, w1d)


def _fin1_body(o_ref, b_ref, out_ref):
    ov = o_ref[...]
    out_ref[...] = (ov[:, :F1] / (ov[:, F1:F1 + 1] + 1e-16)) + b_ref[...]


def _fin1(out1, b1f):
    nb = 10
    rb = N // nb
    return pl.pallas_call(
        _fin1_body,
        grid=(nb,),
        in_specs=[
            pl.BlockSpec((rb, F1 + 128), lambda i: (i, 0)),
            pl.BlockSpec((1, F1), lambda i: (0, 0)),
        ],
        out_specs=pl.BlockSpec((rb, F1), lambda i: (i, 0)),
        out_shape=jax.ShapeDtypeStruct((N, F1), _f32),
    )(out1, b1f)


def kernel(x, edge_index, W0_src, W0_dst, a0, b0, W1_src, W1_dst, a1, b1):
    src = edge_index[0].astype(_i32)
    dst = edge_index[1].astype(_i32)
    a0f = a0.reshape(1, F0)
    b0f = b0.reshape(1, F0)
    a1f = a1.reshape(1, F1)
    b1f = b1.reshape(1, F1)

    fs0, fd0 = _mm2(x, W0_src, W0_dst)
    s0, d0 = _sc_gather(fs0, fd0, src, dst)
    c0 = _edge_math(s0, d0, a0f, H0)
    z0 = jnp.zeros(((64 + 1) * (F0 + 128),), _f32)
    out0 = _sc_scatter(c0, dst, z0, rng=64, sweeps=5)

    fs1, fd1 = _fin0(out0, b0f, W1_src, W1_dst)
    s1, d1 = _sc_gather(fs1, fd1, src, dst)
    c1 = _edge_math(s1, d1, a1f, 1)
    z1 = jnp.zeros(((320 + 1) * (F1 + 128),), _f32)
    out1 = _sc_scatter(c1, dst, z1, rng=320, sweeps=1)

    return _fin1(out1, b1f)


# speedup vs baseline: 1.8161x; 1.1036x over previous
import functools

import jax
import jax.numpy as jnp
from jax import lax
from jax.experimental import pallas as pl
from jax.experimental.pallas import tpu as pltpu
from jax.experimental.pallas import tpu_sc as plsc

N = 10000
E = 160000
IN_DIM = 256
HID = 256
NCLS = 128
H0 = 4
F0 = H0 * HID
F1 = NCLS

NC = 2
NS = 16
NW = NC * NS
EPW = E // NW
GCH = 40
IB = EPW + 64
SCH = 32

_f32 = jnp.float32
_i32 = jnp.int32


def _mesh():
    return plsc.VectorSubcoreMesh(core_axis_name="c", subcore_axis_name="s")


def _mm2_body(x_ref, ws_ref, wd_ref, fs_ref, fd_ref):
    xv = x_ref[...]
    fs_ref[...] = jnp.dot(xv, ws_ref[...], preferred_element_type=_f32)
    fd_ref[...] = jnp.dot(xv, wd_ref[...], preferred_element_type=_f32)


def _mm2(x, ws, wd):
    k, f = x.shape[1], ws.shape[1]
    nb = 10
    rb = N // nb
    return pl.pallas_call(
        _mm2_body,
        grid=(nb,),
        in_specs=[
            pl.BlockSpec((rb, k), lambda i: (i, 0)),
            pl.BlockSpec((k, f), lambda i: (0, 0)),
            pl.BlockSpec((k, f), lambda i: (0, 0)),
        ],
        out_specs=[
            pl.BlockSpec((rb, f), lambda i: (i, 0)),
            pl.BlockSpec((rb, f), lambda i: (i, 0)),
        ],
        out_shape=[
            jax.ShapeDtypeStruct((N, f), _f32),
            jax.ShapeDtypeStruct((N, f), _f32),
        ],
    )(x, ws, wd)


def _sc_gather(fs, fd, src, dst):
    f = fs.shape[1]

    @functools.partial(
        pl.kernel,
        out_type=(
            jax.ShapeDtypeStruct((E, f), _f32),
            jax.ShapeDtypeStruct((E, f), _f32),
        ),
        mesh=_mesh(),
        scratch_types=[
            pltpu.VMEM((GCH,), _i32),
            pltpu.VMEM((GCH,), _i32),
            pltpu.VMEM((GCH, f), _f32),
            pltpu.VMEM((GCH, f), _f32),
            pltpu.SemaphoreType.DMA,
            pltpu.SemaphoreType.DMA,
        ],
    )
    def k(fs_hbm, fd_hbm, src_hbm, dst_hbm, s_hbm, d_hbm,
          idx_s, idx_d, rows_s, rows_d, sem_s, sem_d):
        wid = lax.axis_index("s") * NC + lax.axis_index("c")
        base = wid * EPW

        @pl.loop(0, EPW, step=GCH)
        def _(off):
            o = base + off
            pltpu.sync_copy(src_hbm.at[pl.ds(o, GCH)], idx_s)
            pltpu.sync_copy(dst_hbm.at[pl.ds(o, GCH)], idx_d)
            cs = pltpu.async_copy(fs_hbm.at[idx_s], rows_s, sem_s)
            cd = pltpu.async_copy(fd_hbm.at[idx_d], rows_d, sem_d)
            cs.wait()
            cd.wait()
            pltpu.sync_copy(rows_s, s_hbm.at[pl.ds(o, GCH)])
            pltpu.sync_copy(rows_d, d_hbm.at[pl.ds(o, GCH)])

    return k(fs, fd, src, dst)


def _edge_body(nh, fh, s_ref, d_ref, a_ref, c_ref):
    s = s_ref[...]
    d = d_ref[...]
    a = a_ref[...]
    f = nh * fh
    c_ref[:, f:f + 128] = jnp.zeros_like(c_ref[:, f:f + 128])
    for h in range(nh):
        sh = s[:, h * fh:(h + 1) * fh]
        dh = d[:, h * fh:(h + 1) * fh]
        ah = a[:, h * fh:(h + 1) * fh]
        z = sh + dh
        lr = jnp.where(z > 0, z, 0.2 * z)
        lg = jnp.sum(lr * ah, axis=1, keepdims=True)
        ex = jnp.exp(jnp.minimum(lg, 80.0))
        c_ref[:, h * fh:(h + 1) * fh] = sh * ex
        c_ref[:, f + h:f + h + 1] = ex


def _edge_math(s, d, a_flat, nh):
    f = s.shape[1]
    fh = f // nh
    eb = 640
    nb = E // eb
    return pl.pallas_call(
        functools.partial(_edge_body, nh, fh),
        grid=(nb,),
        in_specs=[
            pl.BlockSpec((eb, f), lambda i: (i, 0)),
            pl.BlockSpec((eb, f), lambda i: (i, 0)),
            pl.BlockSpec((1, f), lambda i: (0, 0)),
        ],
        out_specs=pl.BlockSpec((eb, f + 128), lambda i: (i, 0)),
        out_shape=jax.ShapeDtypeStruct((E, f + 128), _f32),
    )(s, d, a_flat)


NP_ = 10240
ECH = 3200


def _sc_scatter(c, dst, z, rng, sweeps):
    f = c.shape[1]
    accw = (rng + 1) * f
    cp = pltpu.CompilerParams()
    if "needs_layout_passes" in pltpu.CompilerParams.__dataclass_fields__:
        import dataclasses as _dc
        cp = _dc.replace(cp, needs_layout_passes=False)

    @functools.partial(
        pl.kernel,
        out_type=jax.ShapeDtypeStruct((NP_ * f,), _f32),
        mesh=_mesh(),
        compiler_params=cp,
        scratch_types=[
            pltpu.VMEM((ECH,), _i32),
            pltpu.VMEM((ECH + 64,), _i32),
            pltpu.VMEM((ECH + 64,), _i32),
            pltpu.VMEM((16, f), _f32),
            pltpu.VMEM((accw,), _f32),
            pltpu.SemaphoreType.DMA,
        ],
    )
    def k(c_hbm, dst_hbm, z_hbm, out_hbm, dch, ids, lidx, rb0, acc, sem0):
        cid = lax.axis_index("c")
        sid = lax.axis_index("s")
        wid = sid * NC + cid
        iota = lax.iota(_i32, 16)
        zvec = jnp.zeros((16,), _i32)
        tvec = jnp.full((16,), rng, _i32)

        for t in range(sweeps):
            base = (t * NW + wid) * rng
            pltpu.sync_copy(z_hbm, acc)

            @pl.loop(0, E, step=ECH)
            def _(co):
                pltpu.sync_copy(dst_hbm.at[pl.ds(co, ECH)], dch)

                def scan_body(j, cnt):
                    o = j * 64
                    dvs, inrs, incls, ns = [], [], [], []
                    for v in range(4):
                        dv = dch[pl.ds(o + v * 16, 16)]
                        inr = (dv >= base) & (dv < base + rng)
                        dvs.append(dv)
                        inrs.append(inr)
                        incls.append(plsc.cumsum(inr.astype(_i32)))
                        ns.append(plsc.all_reduce_population_count(inr))
                    run = cnt
                    for v in range(4):
                        pos = run + incls[v] - 1
                        plsc.store_scatter(ids, [pos],
                                           co + o + v * 16 + iota,
                                           mask=inrs[v])
                        plsc.store_scatter(lidx, [pos], dvs[v] - base,
                                           mask=inrs[v])
                        run = run + ns[v]
                    return run

                cnt = lax.fori_loop(0, ECH // 64, scan_body,
                                    jnp.zeros((16,), _i32))
                kk = cnt[0]
                for pv in range(3):
                    plsc.store_scatter(ids, [kk + pv * 16 + iota], zvec)
                    plsc.store_scatter(lidx, [kk + pv * 16 + iota], tvec)

                def acc_chunk(off16, rbuf):
                    lvec = lidx[pl.ds(off16, 16)]
                    ros = [lvec[r] * f for r in range(16)]

                    @pl.loop(0, f, step=16)
                    def _(cc):
                        for r in range(16):
                            plsc.addupdate(acc.at[pl.ds(ros[r] + cc, 16)],
                                           rbuf[r, pl.ds(cc, 16)])

                def drain_body(dc, _):
                    pltpu.async_copy(
                        c_hbm.at[ids.at[pl.ds(dc * 16, 16)]], rb0, sem0
                    ).wait()
                    acc_chunk(dc * 16, rb0)
                    return 0

                lax.fori_loop(0, (kk + 15) // 16, drain_body, 0)

            pltpu.sync_copy(acc.at[pl.ds(0, rng * f)],
                            out_hbm.at[pl.ds(base * f, rng * f)])

    return k(c, dst, z).reshape(NP_, f)


def _fin0_body(o_ref, b_ref, ws_ref, wd_ref, fs_ref, fd_ref):
    ov = o_ref[...]
    b = b_ref[...]
    cols = []
    for h in range(H0):
        oh = ov[:, h * HID:(h + 1) * HID]
        bh = b[:, h * HID:(h + 1) * HID]
        dh = ov[:, F0 + h:F0 + h + 1]
        hh = oh / (dh + 1e-16) + bh
        cols.append(jnp.where(hh > 0, hh, jnp.exp(jnp.minimum(hh, 0.0)) - 1.0))
    hv = jnp.concatenate(cols, axis=1)
    fs_ref[...] = jnp.dot(hv, ws_ref[...], preferred_element_type=_f32)
    fd_ref[...] = jnp.dot(hv, wd_ref[...], preferred_element_type=_f32)


def _fin0(out0, b0f, w1s, w1d):
    nb = 10
    rb = N // nb
    return pl.pallas_call(
        _fin0_body,
        grid=(nb,),
        in_specs=[
            pl.BlockSpec((rb, F0 + 128), lambda i: (i, 0)),
            pl.BlockSpec((1, F0), lambda i: (0, 0)),
            pl.BlockSpec((F0, F1), lambda i: (0, 0)),
            pl.BlockSpec((F0, F1), lambda i: (0, 0)),
        ],
        out_specs=[
            pl.BlockSpec((rb, F1), lambda i: (i, 0)),
            pl.BlockSpec((rb, F1), lambda i: (i, 0)),
        ],
        out_shape=[
            jax.ShapeDtypeStruct((N, F1), _f32),
            jax.ShapeDtypeStruct((N, F1), _f32),
        ],
    )(out0, b0f, w1s, w1d)


def _fin1_body(o_ref, b_ref, out_ref):
    ov = o_ref[...]
    out_ref[...] = (ov[:, :F1] / (ov[:, F1:F1 + 1] + 1e-16)) + b_ref[...]


def _fin1(out1, b1f):
    nb = 10
    rb = N // nb
    return pl.pallas_call(
        _fin1_body,
        grid=(nb,),
        in_specs=[
            pl.BlockSpec((rb, F1 + 128), lambda i: (i, 0)),
            pl.BlockSpec((1, F1), lambda i: (0, 0)),
        ],
        out_specs=pl.BlockSpec((rb, F1), lambda i: (i, 0)),
        out_shape=jax.ShapeDtypeStruct((N, F1), _f32),
    )(out1, b1f)


def kernel(x, edge_index, W0_src, W0_dst, a0, b0, W1_src, W1_dst, a1, b1):
    src = edge_index[0].astype(_i32)
    dst = edge_index[1].astype(_i32)
    a0f = a0.reshape(1, F0)
    b0f = b0.reshape(1, F0)
    a1f = a1.reshape(1, F1)
    b1f = b1.reshape(1, F1)

    fs0, fd0 = _mm2(x, W0_src, W0_dst)
    s0, d0 = _sc_gather(fs0, fd0, src, dst)
    c0 = _edge_math(s0, d0, a0f, H0)
    z0 = jnp.zeros(((80 + 1) * (F0 + 128),), _f32)
    out0 = _sc_scatter(c0, dst, z0, rng=80, sweeps=4)

    fs1, fd1 = _fin0(out0, b0f, W1_src, W1_dst)
    s1, d1 = _sc_gather(fs1, fd1, src, dst)
    c1 = _edge_math(s1, d1, a1f, 1)
    z1 = jnp.zeros(((320 + 1) * (F1 + 128),), _f32)
    out1 = _sc_scatter(c1, dst, z1, rng=320, sweeps=1)

    return _fin1(out1, b1f)


# speedup vs baseline: 1.9014x; 1.0469x over previous
import functools

import jax
import jax.numpy as jnp
from jax import lax
from jax.experimental import pallas as pl
from jax.experimental.pallas import tpu as pltpu
from jax.experimental.pallas import tpu_sc as plsc

N = 10000
E = 160000
IN_DIM = 256
HID = 256
NCLS = 128
H0 = 4
F0 = H0 * HID
F1 = NCLS

NC = 2
NS = 16
NW = NC * NS
EPW = E // NW
GCH = 40
IB = EPW + 64
SCH = 32

_f32 = jnp.float32
_i32 = jnp.int32


def _mesh():
    return plsc.VectorSubcoreMesh(core_axis_name="c", subcore_axis_name="s")


def _mm2_body(x_ref, ws_ref, wd_ref, fs_ref, fd_ref):
    xv = x_ref[...]
    fs_ref[...] = jnp.dot(xv, ws_ref[...], preferred_element_type=_f32)
    fd_ref[...] = jnp.dot(xv, wd_ref[...], preferred_element_type=_f32)


def _mm2(x, ws, wd):
    k, f = x.shape[1], ws.shape[1]
    nb = 10
    rb = N // nb
    return pl.pallas_call(
        _mm2_body,
        grid=(nb,),
        in_specs=[
            pl.BlockSpec((rb, k), lambda i: (i, 0)),
            pl.BlockSpec((k, f), lambda i: (0, 0)),
            pl.BlockSpec((k, f), lambda i: (0, 0)),
        ],
        out_specs=[
            pl.BlockSpec((rb, f), lambda i: (i, 0)),
            pl.BlockSpec((rb, f), lambda i: (i, 0)),
        ],
        out_shape=[
            jax.ShapeDtypeStruct((N, f), _f32),
            jax.ShapeDtypeStruct((N, f), _f32),
        ],
    )(x, ws, wd)


def _sc_gather(fs, fd, src, dst):
    f = fs.shape[1]

    @functools.partial(
        pl.kernel,
        out_type=(
            jax.ShapeDtypeStruct((E, f), _f32),
            jax.ShapeDtypeStruct((E, f), _f32),
        ),
        mesh=_mesh(),
        scratch_types=[
            pltpu.VMEM((GCH,), _i32),
            pltpu.VMEM((GCH,), _i32),
            pltpu.VMEM((GCH, f), _f32),
            pltpu.VMEM((GCH, f), _f32),
            pltpu.SemaphoreType.DMA,
            pltpu.SemaphoreType.DMA,
        ],
    )
    def k(fs_hbm, fd_hbm, src_hbm, dst_hbm, s_hbm, d_hbm,
          idx_s, idx_d, rows_s, rows_d, sem_s, sem_d):
        wid = lax.axis_index("s") * NC + lax.axis_index("c")
        base = wid * EPW

        @pl.loop(0, EPW, step=GCH)
        def _(off):
            o = base + off
            i1 = pltpu.async_copy(src_hbm.at[pl.ds(o, GCH)], idx_s, sem_s)
            i2 = pltpu.async_copy(dst_hbm.at[pl.ds(o, GCH)], idx_d, sem_d)
            i1.wait()
            i2.wait()
            cs = pltpu.async_copy(fs_hbm.at[idx_s], rows_s, sem_s)
            cd = pltpu.async_copy(fd_hbm.at[idx_d], rows_d, sem_d)
            cs.wait()
            cd.wait()
            w1 = pltpu.async_copy(rows_s, s_hbm.at[pl.ds(o, GCH)], sem_s)
            w2 = pltpu.async_copy(rows_d, d_hbm.at[pl.ds(o, GCH)], sem_d)
            w1.wait()
            w2.wait()

    return k(fs, fd, src, dst)


def _edge_body(nh, fh, s_ref, d_ref, a_ref, c_ref):
    s = s_ref[...]
    d = d_ref[...]
    a = a_ref[...]
    f = nh * fh
    c_ref[:, f:f + 128] = jnp.zeros_like(c_ref[:, f:f + 128])
    for h in range(nh):
        sh = s[:, h * fh:(h + 1) * fh]
        dh = d[:, h * fh:(h + 1) * fh]
        ah = a[:, h * fh:(h + 1) * fh]
        z = sh + dh
        lr = jnp.where(z > 0, z, 0.2 * z)
        lg = jnp.sum(lr * ah, axis=1, keepdims=True)
        ex = jnp.exp(jnp.minimum(lg, 80.0))
        c_ref[:, h * fh:(h + 1) * fh] = sh * ex
        c_ref[:, f + h:f + h + 1] = ex


def _edge_math(s, d, a_flat, nh):
    f = s.shape[1]
    fh = f // nh
    eb = 640
    nb = E // eb
    return pl.pallas_call(
        functools.partial(_edge_body, nh, fh),
        grid=(nb,),
        in_specs=[
            pl.BlockSpec((eb, f), lambda i: (i, 0)),
            pl.BlockSpec((eb, f), lambda i: (i, 0)),
            pl.BlockSpec((1, f), lambda i: (0, 0)),
        ],
        out_specs=pl.BlockSpec((eb, f + 128), lambda i: (i, 0)),
        out_shape=jax.ShapeDtypeStruct((E, f + 128), _f32),
    )(s, d, a_flat)


NP_ = 10240
ECH = 3200


def _sc_scatter(c, dst, z, rng, sweeps):
    f = c.shape[1]
    accw = (rng + 1) * f
    cp = pltpu.CompilerParams()
    if "needs_layout_passes" in pltpu.CompilerParams.__dataclass_fields__:
        import dataclasses as _dc
        cp = _dc.replace(cp, needs_layout_passes=False)

    @functools.partial(
        pl.kernel,
        out_type=jax.ShapeDtypeStruct((NP_ * f,), _f32),
        mesh=_mesh(),
        compiler_params=cp,
        scratch_types=[
            pltpu.VMEM((ECH,), _i32),
            pltpu.VMEM((ECH,), _i32),
            pltpu.VMEM((ECH + 64,), _i32),
            pltpu.VMEM((ECH + 64,), _i32),
            pltpu.VMEM((16, f), _f32),
            pltpu.VMEM((accw,), _f32),
            pltpu.SemaphoreType.DMA,
            pltpu.SemaphoreType.DMA,
        ],
    )
    def k(c_hbm, dst_hbm, z_hbm, out_hbm, dch0, dch1, ids, lidx, rb0, acc,
          sem0, semd):
        cid = lax.axis_index("c")
        sid = lax.axis_index("s")
        wid = sid * NC + cid
        iota = lax.iota(_i32, 16)
        zvec = jnp.zeros((16,), _i32)
        tvec = jnp.full((16,), rng, _i32)

        for t in range(sweeps):
            base = (t * NW + wid) * rng
            pltpu.sync_copy(z_hbm, acc)

            def process(dch, co):
                def scan_body(j, cnt):
                    o = j * 64
                    dvs, inrs, incls, ns = [], [], [], []
                    for v in range(4):
                        dv = dch[pl.ds(o + v * 16, 16)]
                        inr = (dv >= base) & (dv < base + rng)
                        dvs.append(dv)
                        inrs.append(inr)
                        incls.append(plsc.cumsum(inr.astype(_i32)))
                        ns.append(plsc.all_reduce_population_count(inr))
                    run = cnt
                    for v in range(4):
                        pos = run + incls[v] - 1
                        plsc.store_scatter(ids, [pos],
                                           co + o + v * 16 + iota,
                                           mask=inrs[v])
                        plsc.store_scatter(lidx, [pos], dvs[v] - base,
                                           mask=inrs[v])
                        run = run + ns[v]
                    return run

                cnt = lax.fori_loop(0, ECH // 64, scan_body,
                                    jnp.zeros((16,), _i32))
                kk = cnt[0]
                for pv in range(3):
                    plsc.store_scatter(ids, [kk + pv * 16 + iota], zvec)
                    plsc.store_scatter(lidx, [kk + pv * 16 + iota], tvec)

                def acc_chunk(off16, rbuf):
                    lvec = lidx[pl.ds(off16, 16)]
                    ros = [lvec[r] * f for r in range(16)]

                    @pl.loop(0, f, step=16)
                    def _(cc):
                        for r in range(16):
                            plsc.addupdate(acc.at[pl.ds(ros[r] + cc, 16)],
                                           rbuf[r, pl.ds(cc, 16)])

                def drain_body(dc, _):
                    pltpu.async_copy(
                        c_hbm.at[ids.at[pl.ds(dc * 16, 16)]], rb0, sem0
                    ).wait()
                    acc_chunk(dc * 16, rb0)
                    return 0

                lax.fori_loop(0, (kk + 15) // 16, drain_body, 0)

            nch = E // ECH
            pltpu.sync_copy(dst_hbm.at[pl.ds(0, ECH)], dch0)

            def chunk_pair(q, _):
                c0 = q * 2 * ECH
                d1 = pltpu.async_copy(
                    dst_hbm.at[pl.ds(c0 + ECH, ECH)], dch1, semd)
                process(dch0, c0)
                d1.wait()
                c2 = lax.rem(c0 + 2 * ECH, E)
                d0 = pltpu.async_copy(dst_hbm.at[pl.ds(c2, ECH)], dch0, semd)
                process(dch1, c0 + ECH)
                d0.wait()
                return 0

            lax.fori_loop(0, nch // 2, chunk_pair, 0)
            pltpu.sync_copy(acc.at[pl.ds(0, rng * f)],
                            out_hbm.at[pl.ds(base * f, rng * f)])

    return k(c, dst, z).reshape(NP_, f)


def _fin0_body(o_ref, b_ref, ws_ref, wd_ref, fs_ref, fd_ref):
    ov = o_ref[...]
    b = b_ref[...]
    cols = []
    for h in range(H0):
        oh = ov[:, h * HID:(h + 1) * HID]
        bh = b[:, h * HID:(h + 1) * HID]
        dh = ov[:, F0 + h:F0 + h + 1]
        hh = oh / (dh + 1e-16) + bh
        cols.append(jnp.where(hh > 0, hh, jnp.exp(jnp.minimum(hh, 0.0)) - 1.0))
    hv = jnp.concatenate(cols, axis=1)
    fs_ref[...] = jnp.dot(hv, ws_ref[...], preferred_element_type=_f32)
    fd_ref[...] = jnp.dot(hv, wd_ref[...], preferred_element_type=_f32)


def _fin0(out0, b0f, w1s, w1d):
    nb = 10
    rb = N // nb
    return pl.pallas_call(
        _fin0_body,
        grid=(nb,),
        in_specs=[
            pl.BlockSpec((rb, F0 + 128), lambda i: (i, 0)),
            pl.BlockSpec((1, F0), lambda i: (0, 0)),
            pl.BlockSpec((F0, F1), lambda i: (0, 0)),
            pl.BlockSpec((F0, F1), lambda i: (0, 0)),
        ],
        out_specs=[
            pl.BlockSpec((rb, F1), lambda i: (i, 0)),
            pl.BlockSpec((rb, F1), lambda i: (i, 0)),
        ],
        out_shape=[
            jax.ShapeDtypeStruct((N, F1), _f32),
            jax.ShapeDtypeStruct((N, F1), _f32),
        ],
    )(out0, b0f, w1s, w1d)


def _fin1_body(o_ref, b_ref, out_ref):
    ov = o_ref[...]
    out_ref[...] = (ov[:, :F1] / (ov[:, F1:F1 + 1] + 1e-16)) + b_ref[...]


def _fin1(out1, b1f):
    nb = 10
    rb = N // nb
    return pl.pallas_call(
        _fin1_body,
        grid=(nb,),
        in_specs=[
            pl.BlockSpec((rb, F1 + 128), lambda i: (i, 0)),
            pl.BlockSpec((1, F1), lambda i: (0, 0)),
        ],
        out_specs=pl.BlockSpec((rb, F1), lambda i: (i, 0)),
        out_shape=jax.ShapeDtypeStruct((N, F1), _f32),
    )(out1, b1f)


def kernel(x, edge_index, W0_src, W0_dst, a0, b0, W1_src, W1_dst, a1, b1):
    src = edge_index[0].astype(_i32)
    dst = edge_index[1].astype(_i32)
    a0f = a0.reshape(1, F0)
    b0f = b0.reshape(1, F0)
    a1f = a1.reshape(1, F1)
    b1f = b1.reshape(1, F1)

    fs0, fd0 = _mm2(x, W0_src, W0_dst)
    s0, d0 = _sc_gather(fs0, fd0, src, dst)
    c0 = _edge_math(s0, d0, a0f, H0)
    z0 = jnp.zeros(((80 + 1) * (F0 + 128),), _f32)
    out0 = _sc_scatter(c0, dst, z0, rng=80, sweeps=4)

    fs1, fd1 = _fin0(out0, b0f, W1_src, W1_dst)
    s1, d1 = _sc_gather(fs1, fd1, src, dst)
    c1 = _edge_math(s1, d1, a1f, 1)
    z1 = jnp.zeros(((320 + 1) * (F1 + 128),), _f32)
    out1 = _sc_scatter(c1, dst, z1, rng=320, sweeps=1)

    return _fin1(out1, b1f)
